# Initial kernel scaffold; baseline (speedup 1.0000x reference)
import jax
import jax.numpy as jnp
from jax.experimental import pallas as pl


def kernel(atom_idx, atom_charges, motif_features, shape_classes, mult_per_atom, atom_id_table, atom_charge_table, shape_id_table, atom_mult_table, W_motif, b_motif):
    raise NotImplementedError("write your pallas kernel here")


# speedup vs baseline: 2.7091x; 2.7091x over previous
import functools

import jax
import jax.numpy as jnp
from jax import lax
from jax.experimental import pallas as pl
from jax.experimental.pallas import tpu as pltpu
from jax.experimental.pallas import tpu_sc as plsc

N = 100000
ATOM_ID_DIM = 32
CHARGE_DIM = 8
SHAPE_ID_DIM = 16
MULT_DIM = 16
MOTIF_FEAT_SIZE = 48
MOTIF_DIM = 32
NUM_JOINS = 3
OUT_DIM = ATOM_ID_DIM + CHARGE_DIM + NUM_JOINS * (MOTIF_DIM + SHAPE_ID_DIM + MULT_DIM)


CHUNK = 1000
NUM_CHUNKS = N // CHUNK


def _sc_gather(atom_idx, sidx, atom_tab, shape_tab):
    info = plsc.get_sparse_core_info()
    nc, ns = info.num_cores, info.num_subcores
    nw = nc * ns
    chunks_per_w = -(-NUM_CHUNKS // nw)
    mesh = plsc.VectorSubcoreMesh(core_axis_name="c", subcore_axis_name="s")

    @functools.partial(
        pl.kernel,
        mesh=mesh,
        out_type=(
            jax.ShapeDtypeStruct((N, ATOM_ID_DIM), jnp.float32),
            jax.ShapeDtypeStruct((NUM_JOINS * N, SHAPE_ID_DIM), jnp.float32),
        ),
        scratch_types=[
            pltpu.VMEM((CHUNK,), jnp.int32),
            pltpu.VMEM((CHUNK, ATOM_ID_DIM), jnp.float32),
            pltpu.VMEM((CHUNK,), jnp.int32),
            pltpu.VMEM((CHUNK, SHAPE_ID_DIM), jnp.float32),
            pltpu.SemaphoreType.DMA,
        ],
        compiler_params=pltpu.CompilerParams(use_tc_tiling_on_sc=False),
    )
    def k(aidx_hbm, sidx_hbm, atab_hbm, stab_hbm, out_a, out_s,
          aidx_v, arows_v, sidx_v, srows_v, sem):
        wid = lax.axis_index("s") * nc + lax.axis_index("c")
        for c in range(chunks_per_w):
            cid = wid + nw * c

            @pl.when(cid < NUM_CHUNKS)
            def _():
                base = cid * CHUNK
                pltpu.sync_copy(aidx_hbm.at[pl.ds(base, CHUNK)], aidx_v)
                pltpu.async_copy(atab_hbm.at[aidx_v], arows_v, sem).wait()
                pltpu.sync_copy(arows_v, out_a.at[pl.ds(base, CHUNK)])
                for j in range(NUM_JOINS):
                    pltpu.sync_copy(sidx_hbm.at[pl.ds(j * N + base, CHUNK)], sidx_v)
                    pltpu.async_copy(stab_hbm.at[sidx_v], srows_v, sem).wait()
                    pltpu.sync_copy(srows_v, out_s.at[pl.ds(j * N + base, CHUNK)])

    return k(atom_idx, sidx, atom_tab, shape_tab)


BR = 1000


def _tc_body(motif_ref, ch_ref, mult_ref, ga_ref, gs0_ref, gs1_ref, gs2_ref,
             ctab_ref, mtab_ref, w_ref, b_ref, out_ref):
    gs_refs = (gs0_ref, gs1_ref, gs2_ref)
    out_ref[:, 0:ATOM_ID_DIM] = ga_ref[...]

    ch = ch_ref[...] + 1
    oh_c = (ch == lax.broadcasted_iota(jnp.int32, (BR, 8), 1)).astype(jnp.float32)
    out_ref[:, 32:40] = jnp.dot(oh_c, ctab_ref[...],
                                preferred_element_type=jnp.float32,
                                precision=lax.Precision.HIGHEST)

    w = w_ref[...]
    b = b_ref[...]
    mtab = mtab_ref[...]
    for j in range(NUM_JOINS):
        mf = motif_ref[:, j * MOTIF_FEAT_SIZE:(j + 1) * MOTIF_FEAT_SIZE]
        r = jnp.dot(mf, w, preferred_element_type=jnp.float32,
                    precision=lax.Precision.HIGHEST) + b
        out_ref[:, 40 + 32 * j:72 + 32 * j] = r

        out_ref[:, 136 + 16 * j:152 + 16 * j] = gs_refs[j][...]

        m = mult_ref[:, j:j + 1] + 1
        oh_m = (m == lax.broadcasted_iota(jnp.int32, (BR, 32), 1)).astype(jnp.float32)
        out_ref[:, 184 + 16 * j:200 + 16 * j] = jnp.dot(
            oh_m, mtab, preferred_element_type=jnp.float32,
            precision=lax.Precision.HIGHEST)


def _tc_assemble(motif, charges, mult, gath_a, gath_s, ctab_pad, mtab, w, b):
    grid = (N // BR,)
    return pl.pallas_call(
        _tc_body,
        grid=grid,
        in_specs=[
            pl.BlockSpec((BR, NUM_JOINS * MOTIF_FEAT_SIZE), lambda i: (i, 0)),
            pl.BlockSpec((BR, 1), lambda i: (i, 0)),
            pl.BlockSpec((BR, NUM_JOINS), lambda i: (i, 0)),
            pl.BlockSpec((BR, ATOM_ID_DIM), lambda i: (i, 0)),
            pl.BlockSpec((BR, SHAPE_ID_DIM), lambda i: (i, 0)),
            pl.BlockSpec((BR, SHAPE_ID_DIM), lambda i: (N // BR + i, 0)),
            pl.BlockSpec((BR, SHAPE_ID_DIM), lambda i: (2 * (N // BR) + i, 0)),
            pl.BlockSpec((8, CHARGE_DIM), lambda i: (0, 0)),
            pl.BlockSpec((32, MULT_DIM), lambda i: (0, 0)),
            pl.BlockSpec((MOTIF_FEAT_SIZE, MOTIF_DIM), lambda i: (0, 0)),
            pl.BlockSpec((1, MOTIF_DIM), lambda i: (0, 0)),
        ],
        out_specs=pl.BlockSpec((BR, OUT_DIM), lambda i: (i, 0)),
        out_shape=jax.ShapeDtypeStruct((N, OUT_DIM), jnp.float32),
        compiler_params=pltpu.CompilerParams(
            dimension_semantics=("arbitrary",),
        ),
    )(motif, charges, mult, gath_a, gath_s, gath_s, gath_s, ctab_pad, mtab, w, b)


def kernel(atom_idx, atom_charges, motif_features, shape_classes, mult_per_atom,
           atom_id_table, atom_charge_table, shape_id_table, atom_mult_table,
           W_motif, b_motif):
    sidx = (shape_classes.astype(jnp.int32) + 1).T.reshape(-1)
    gath_a, gath_s = _sc_gather(atom_idx.astype(jnp.int32), sidx,
                                atom_id_table, shape_id_table)
    ctab_pad = jnp.zeros((8, CHARGE_DIM), jnp.float32).at[:3].set(atom_charge_table)
    return _tc_assemble(
        motif_features,
        atom_charges.astype(jnp.int32).reshape(N, 1),
        mult_per_atom.astype(jnp.int32),
        gath_a, gath_s, ctab_pad, atom_mult_table, W_motif,
        b_motif.reshape(1, MOTIF_DIM),
    )
